# baseline (speedup 1.0000x reference)
import functools

import jax
import jax.numpy as jnp
from jax import lax
from jax.experimental import pallas as pl
from jax.experimental.pallas import tpu as pltpu
from jax.experimental.pallas import tpu_sc as plsc

L = 16
NC = 2
NS = 16
NW = NC * NS
V = 10

B, SEQ = 16384, 200
COLS_W = B // NW
CW = 2048
NBIG = 6


@functools.partial(
    pl.kernel,
    out_type=jax.ShapeDtypeStruct((NW, L), jnp.float32),
    mesh=plsc.VectorSubcoreMesh(core_axis_name="c", subcore_axis_name="s"),
    compiler_params=pltpu.CompilerParams(needs_layout_passes=False),
    scratch_types=[
        pltpu.VMEM((8, CW), jnp.int32),
        pltpu.VMEM((8, CW), jnp.int32),
        pltpu.VMEM((8, COLS_W), jnp.int32),
        pltpu.VMEM((V, V), jnp.float32),
        pltpu.VMEM((V, V), jnp.float32),
        pltpu.VMEM((L * L,), jnp.float32),
        pltpu.VMEM((1, L), jnp.float32),
        pltpu.SemaphoreType.DMA,
        pltpu.SemaphoreType.DMA,
        pltpu.SemaphoreType.DMA,
    ],
)
def _sc_sum(xt_hbm, ea_hbm, eb_hbm, out_hbm,
            buf0, buf1, bufz, tab_a, tab_b, pair, acc_ref,
            sem0, sem1, semz):
    cid = lax.axis_index("c")
    sid = lax.axis_index("s")
    wid = sid * NC + cid
    col0 = wid * COLS_W

    pltpu.sync_copy(ea_hbm, tab_a)
    pltpu.sync_copy(eb_hbm, tab_b)
    rows = lax.iota(jnp.int32, L)
    keep = rows < V
    zeros = jnp.zeros((L,), jnp.float32)
    s = zeros
    for k in range(V):
        col = jnp.full((L,), k, jnp.int32)
        s = s + plsc.load_gather(tab_a, [rows, col], mask=keep)
        s = s + plsc.load_gather(tab_b, [rows, col], mask=keep)
    s = jnp.where(keep, s, zeros)

    for a in range(V):
        pair[pl.ds(a * L, L)] = s[a] + s

    bufs = (buf0, buf1)
    sems = (sem0, sem1)

    def dma(k, buf, sem):
        t = 96 * wid + 16 * k
        i = t // 128
        j = t % 128
        return pltpu.make_async_copy(
            xt_hbm.at[pl.ds(8 * i, 8), pl.ds(128 * j, CW)], buf, sem)

    zcopy = pltpu.make_async_copy(
        xt_hbm.at[pl.ds(SEQ - 8, 8), pl.ds(col0, COLS_W)], bufz, semz)

    dma(0, buf0, sem0).start()
    zcopy.start()

    def accum(buf, width, a):
        def body(r, a, buf=buf, width=width):
            for j in range(width // (2 * L)):
                ia = buf[r, pl.ds(2 * j * L, L)]
                ib = buf[r, pl.ds((2 * j + 1) * L, L)]
                a = a + plsc.load_gather(pair, [ia * L + ib])
            return a
        return lax.fori_loop(0, 8, body, a)

    acc = zeros
    for c in range(NBIG):
        buf, sem = bufs[c % 2], sems[c % 2]
        if c + 1 < NBIG:
            dma(c + 1, bufs[(c + 1) % 2], sems[(c + 1) % 2]).start()
        dma(c, buf, sem).wait()
        acc = accum(buf, CW, acc)

    zcopy.wait()
    acc = accum(bufz, COLS_W, acc)

    acc_ref[0, :] = acc
    pltpu.sync_copy(acc_ref, out_hbm.at[pl.ds(wid, 1)])


def kernel(x, emb, emb2):
    xt = x.astype(jnp.int32).T
    partials = _sc_sum(xt, emb, emb2)
    return jnp.sum(partials)
